# Initial kernel scaffold; baseline (speedup 1.0000x reference)
import jax
import jax.numpy as jnp
from jax.experimental import pallas as pl


def kernel(x, weight):
    raise NotImplementedError("write your pallas kernel here")


# speedup vs baseline: 1.7968x; 1.7968x over previous
import functools

import jax
import jax.numpy as jnp
from jax import lax
from jax.experimental import pallas as pl
from jax.experimental.pallas import tpu as pltpu
from jax.experimental.pallas import tpu_sc as plsc

VOCAB = 1000000
DIM = 64
ROWS = 16384
COLS = 50
N = ROWS * COLS
NC, NS = 2, 16
NW = NC * NS
PER_W = N // NW
IDX_W = 128
SUB = 4
CHUNK = IDX_W * SUB
N_CHUNKS = PER_W // CHUNK

_MESH = plsc.VectorSubcoreMesh(
    core_axis_name="c", subcore_axis_name="s", num_cores=NC, num_subcores=NS
)


@functools.partial(
    pl.kernel,
    out_type=jax.ShapeDtypeStruct((N, DIM), jnp.float32),
    mesh=_MESH,
    scratch_types=[
        pltpu.VMEM((SUB, IDX_W), jnp.int32),
        pltpu.VMEM((CHUNK, DIM), jnp.float32),
        pltpu.SemaphoreType.DMA,
    ],
    compiler_params=pltpu.CompilerParams(use_tc_tiling_on_sc=False),
)
def _gather_kernel(x_hbm, w_hbm, out_hbm, idx_v, rows_v, sem):
    wid = lax.axis_index("s") * NC + lax.axis_index("c")
    row0 = wid * (PER_W // IDX_W)

    def chunk(j, _):
        pltpu.sync_copy(x_hbm.at[pl.ds(row0 + j * SUB, SUB)], idx_v)
        descs = [
            pltpu.async_copy(
                w_hbm.at[idx_v.at[b]],
                rows_v.at[pl.ds(b * IDX_W, IDX_W)],
                sem,
            )
            for b in range(SUB)
        ]
        for d in descs:
            d.wait()
        pltpu.sync_copy(
            rows_v, out_hbm.at[pl.ds(wid * PER_W + j * CHUNK, CHUNK)]
        )
        return 0

    lax.fori_loop(0, N_CHUNKS, chunk, 0)


def kernel(x, weight):
    x2d = x.reshape(N // IDX_W, IDX_W).astype(jnp.int32)
    out = _gather_kernel(x2d, weight)
    return out.reshape(ROWS, COLS, DIM)


# speedup vs baseline: 1.8538x; 1.0317x over previous
import functools

import jax
import jax.numpy as jnp
from jax import lax
from jax.experimental import pallas as pl
from jax.experimental.pallas import tpu as pltpu
from jax.experimental.pallas import tpu_sc as plsc

VOCAB = 1000000
DIM = 64
ROWS = 16384
COLS = 50
N = ROWS * COLS
NC, NS = 2, 16
NW = NC * NS
PER_W = N // NW
IDX_W = 128
SUB = 4
CHUNK = IDX_W * SUB
N_CHUNKS = PER_W // CHUNK

_MESH = plsc.VectorSubcoreMesh(
    core_axis_name="c", subcore_axis_name="s", num_cores=NC, num_subcores=NS
)


@functools.partial(
    pl.kernel,
    out_type=jax.ShapeDtypeStruct((N, DIM), jnp.float32),
    mesh=_MESH,
    scratch_types=[
        pltpu.VMEM((SUB, IDX_W), jnp.int32),
        pltpu.VMEM((SUB, IDX_W), jnp.int32),
        pltpu.VMEM((CHUNK, DIM), jnp.float32),
        pltpu.VMEM((CHUNK, DIM), jnp.float32),
        pltpu.SemaphoreType.DMA,
        pltpu.SemaphoreType.DMA,
        pltpu.SemaphoreType.DMA,
        pltpu.SemaphoreType.DMA,
    ],
    compiler_params=pltpu.CompilerParams(use_tc_tiling_on_sc=False),
)
def _gather_kernel(x_hbm, w_hbm, out_hbm, idx0, idx1, rows0, rows1,
                   gsem0, gsem1, ssem0, ssem1):
    wid = lax.axis_index("s") * NC + lax.axis_index("c")
    row0 = wid * (PER_W // IDX_W)
    out0 = wid * PER_W

    def idx_load(g, idx_v):
        pltpu.sync_copy(x_hbm.at[pl.ds(row0 + g * SUB, SUB)], idx_v)

    def fire_g(idx_v, rows_v, sem):
        for b in range(SUB):
            pltpu.async_copy(
                w_hbm.at[idx_v.at[b]],
                rows_v.at[pl.ds(b * IDX_W, IDX_W)],
                sem,
            )

    def wait_g(rows_v, sem):
        pltpu.make_async_copy(w_hbm.at[pl.ds(0, CHUNK)], rows_v, sem).wait()

    def fire_s(g, rows_v, sem):
        pltpu.async_copy(rows_v, out_hbm.at[pl.ds(out0 + g * CHUNK, CHUNK)], sem)

    def wait_s(rows_v, sem):
        pltpu.make_async_copy(rows_v, out_hbm.at[pl.ds(0, CHUNK)], sem).wait()

    idx_load(0, idx0)
    fire_g(idx0, rows0, gsem0)

    def pair(i, _):
        j = i * 2

        @pl.when(i > 0)
        def _():
            wait_s(rows1, ssem1)

        idx_load(j + 1, idx1)
        fire_g(idx1, rows1, gsem1)

        wait_g(rows0, gsem0)
        fire_s(j, rows0, ssem0)

        @pl.when(j + 2 < N_CHUNKS)
        def _():
            wait_s(rows0, ssem0)
            idx_load(j + 2, idx0)
            fire_g(idx0, rows0, gsem0)

        wait_g(rows1, gsem1)
        fire_s(j + 1, rows1, ssem1)
        return 0

    lax.fori_loop(0, N_CHUNKS // 2, pair, 0)

    wait_s(rows0, ssem0)
    wait_s(rows1, ssem1)


def kernel(x, weight):
    x2d = x.reshape(N // IDX_W, IDX_W).astype(jnp.int32)
    out = _gather_kernel(x2d, weight)
    return out.reshape(ROWS, COLS, DIM)
